# baseline (speedup 1.0000x reference)
import functools
import math

import jax
import jax.numpy as jnp
from jax.experimental import pallas as pl

N_HEADS = 16
_FACTOR = 5
_N_ITERS = 18


def _proj_kernel(x_ref, w_ref, b_ref, o_ref):
    x = x_ref[...]
    w = w_ref[...]
    acc = jax.lax.dot_general(x, w, (((1,), (1,)), ((), ())),
                              preferred_element_type=jnp.float32)
    o_ref[...] = (acc + b_ref[0])[None]


def _project(x, W, b, n_blk):
    N, D = x.shape
    H = N_HEADS
    d = D // H
    b3 = b.reshape(H, 1, d)
    return pl.pallas_call(
        _proj_kernel,
        grid=(H, N // n_blk),
        in_specs=[
            pl.BlockSpec((n_blk, D), lambda h, n: (n, 0)),
            pl.BlockSpec((d, D), lambda h, n: (h, 0)),
            pl.BlockSpec((1, 1, d), lambda h, n: (h, 0, 0)),
        ],
        out_specs=pl.BlockSpec((1, n_blk, d), lambda h, n: (h, n, 0)),
        out_shape=jax.ShapeDtypeStruct((H, N, d), jnp.float32),
    )(x, W, b3)


def _attn_kernel(q_ref, k_ref, v_ref, o_ref, *, U, n_iters):
    q = q_ref[0]
    k = k_ref[0]
    v = v_ref[0]
    s = jax.lax.dot_general(q, k, (((1,), (1,)), ((), ())),
                            preferred_element_type=jnp.float32)
    m = jnp.max(s, axis=-1, keepdims=True)
    lo0 = jnp.min(s, axis=-1, keepdims=True)
    kcnt = jnp.float32(U)
    ones = jnp.ones((s.shape[1], 128), jnp.float32)

    def body(_, carry):
        lo, hi = carry
        mid = 0.5 * (lo + hi)
        mask = (s >= mid).astype(jnp.float32)
        cnt = jax.lax.dot_general(mask, ones, (((1,), (0,)), ((), ())),
                                  preferred_element_type=jnp.float32)[:, :1]
        pred = cnt >= kcnt
        return jnp.where(pred, mid, lo), jnp.where(pred, hi, mid)

    lo, _ = jax.lax.fori_loop(0, n_iters, body, (lo0, m))
    mprime = jnp.maximum(m, 0.0)
    bg = jnp.exp(-mprime)
    p = jnp.where(s >= lo, jnp.exp(s - mprime), bg)
    z = jax.lax.dot_general(p, ones, (((1,), (0,)), ((), ())),
                            preferred_element_type=jnp.float32)[:, :1]
    o = jax.lax.dot_general(p, v, (((1,), (0,)), ((), ())),
                            preferred_element_type=jnp.float32)
    o_ref[...] = (o / z)[None]


def _attention(q, k, v, U, l_blk):
    H, L, d = q.shape
    S = k.shape[1]
    return pl.pallas_call(
        functools.partial(_attn_kernel, U=U, n_iters=_N_ITERS),
        grid=(H, L // l_blk),
        in_specs=[
            pl.BlockSpec((1, l_blk, d), lambda h, l: (h, l, 0)),
            pl.BlockSpec((1, S, d), lambda h, l: (h, 0, 0)),
            pl.BlockSpec((1, S, d), lambda h, l: (h, 0, 0)),
        ],
        out_specs=pl.BlockSpec((1, l_blk, d), lambda h, l: (h, l, 0)),
        out_shape=jax.ShapeDtypeStruct((H, L, d), jnp.float32),
    )(q, k, v)


def kernel(queries, keys, values, Wq, bq, Wk, bk, Wv, bv):
    B_, L, D = queries.shape
    S = keys.shape[1]
    U = int(_FACTOR * math.log(L))
    n_blk = min(256, L)
    q = _project(queries.reshape(B_ * L, D), Wq, bq, n_blk)
    k = _project(keys.reshape(B_ * S, D), Wk, bk, n_blk)
    v = _project(values.reshape(B_ * S, D), Wv, bv, n_blk)
    out = _attention(q, k, v, U, n_blk)
    return out.transpose(1, 0, 2).reshape(B_, L, D)


# speedup vs baseline: 1.2646x; 1.2646x over previous
import functools
import math

import jax
import jax.numpy as jnp
from jax.experimental import pallas as pl

N_HEADS = 16
_FACTOR = 5
_N_ITERS = 12


def _proj_kernel(x_ref, w_ref, b_ref, o_ref):
    x = x_ref[...]
    w = w_ref[...]
    acc = jax.lax.dot_general(x, w, (((1,), (1,)), ((), ())),
                              preferred_element_type=jnp.float32)
    o_ref[...] = (acc + b_ref[0])[None]


def _project(x, W, b, n_blk):
    N, D = x.shape
    H = N_HEADS
    d = D // H
    b3 = b.reshape(H, 1, d)
    return pl.pallas_call(
        _proj_kernel,
        grid=(H, N // n_blk),
        in_specs=[
            pl.BlockSpec((n_blk, D), lambda h, n: (n, 0)),
            pl.BlockSpec((d, D), lambda h, n: (h, 0)),
            pl.BlockSpec((1, 1, d), lambda h, n: (h, 0, 0)),
        ],
        out_specs=pl.BlockSpec((1, n_blk, d), lambda h, n: (h, n, 0)),
        out_shape=jax.ShapeDtypeStruct((H, N, d), jnp.float32),
    )(x, W, b3)


def _attn_kernel(q_ref, k_ref, v_ref, o_ref, *, U, n_iters):
    q = q_ref[0]
    k = k_ref[0]
    v = v_ref[0]
    s = jax.lax.dot_general(q, k, (((1,), (1,)), ((), ())),
                            preferred_element_type=jnp.float32)
    m = jnp.max(s, axis=-1, keepdims=True)
    lo0 = jnp.min(s, axis=-1, keepdims=True)
    kcnt = jnp.float32(U)
    ones = jnp.ones((s.shape[1], 128), jnp.float32)

    def body(_, carry):
        lo, hi = carry
        mid = 0.5 * (lo + hi)
        mask = (s >= mid).astype(jnp.float32)
        cnt = jax.lax.dot_general(mask, ones, (((1,), (0,)), ((), ())),
                                  preferred_element_type=jnp.float32)[:, :1]
        pred = cnt >= kcnt
        return jnp.where(pred, mid, lo), jnp.where(pred, hi, mid)

    lo, _ = jax.lax.fori_loop(0, n_iters, body, (lo0, m))
    mprime = jnp.maximum(m, 0.0)
    bg = jnp.exp(-mprime)
    p = jnp.where(s >= lo, jnp.exp(s - mprime), bg)
    z = jax.lax.dot_general(p, ones, (((1,), (0,)), ((), ())),
                            preferred_element_type=jnp.float32)[:, :1]
    o = jax.lax.dot_general(p, v, (((1,), (0,)), ((), ())),
                            preferred_element_type=jnp.float32)
    o_ref[...] = (o / z)[None]


def _attention(q, k, v, U, l_blk):
    H, L, d = q.shape
    S = k.shape[1]
    return pl.pallas_call(
        functools.partial(_attn_kernel, U=U, n_iters=_N_ITERS),
        grid=(H, L // l_blk),
        in_specs=[
            pl.BlockSpec((1, l_blk, d), lambda h, l: (h, l, 0)),
            pl.BlockSpec((1, S, d), lambda h, l: (h, 0, 0)),
            pl.BlockSpec((1, S, d), lambda h, l: (h, 0, 0)),
        ],
        out_specs=pl.BlockSpec((1, l_blk, d), lambda h, l: (h, l, 0)),
        out_shape=jax.ShapeDtypeStruct((H, L, d), jnp.float32),
    )(q, k, v)


def kernel(queries, keys, values, Wq, bq, Wk, bk, Wv, bv):
    B_, L, D = queries.shape
    S = keys.shape[1]
    U = int(_FACTOR * math.log(L))
    n_blk = min(256, L)
    q = _project(queries.reshape(B_ * L, D), Wq, bq, n_blk)
    k = _project(keys.reshape(B_ * S, D), Wk, bk, n_blk)
    v = _project(values.reshape(B_ * S, D), Wv, bv, n_blk)
    out = _attention(q, k, v, U, n_blk)
    return out.transpose(1, 0, 2).reshape(B_, L, D)


# speedup vs baseline: 1.6142x; 1.2765x over previous
import functools
import math

import jax
import jax.numpy as jnp
from jax.experimental import pallas as pl

N_HEADS = 16
_FACTOR = 5
_N_ITERS = 11


def _proj_kernel(x_ref, w_ref, b_ref, o_ref):
    x = x_ref[...]
    w = w_ref[...]
    acc = jax.lax.dot_general(x, w, (((1,), (1,)), ((), ())),
                              preferred_element_type=jnp.float32)
    o_ref[...] = (acc + b_ref[0])[None]


def _project(x, W, b, n_blk):
    N, D = x.shape
    H = N_HEADS
    d = D // H
    b3 = b.reshape(H, 1, d)
    return pl.pallas_call(
        _proj_kernel,
        grid=(H, N // n_blk),
        in_specs=[
            pl.BlockSpec((n_blk, D), lambda h, n: (n, 0)),
            pl.BlockSpec((d, D), lambda h, n: (h, 0)),
            pl.BlockSpec((1, 1, d), lambda h, n: (h, 0, 0)),
        ],
        out_specs=pl.BlockSpec((1, n_blk, d), lambda h, n: (h, n, 0)),
        out_shape=jax.ShapeDtypeStruct((H, N, d), jnp.float32),
    )(x, W, b3)


def _attn_kernel(q_ref, k_ref, v_ref, o_ref, *, U, n_iters):
    q = q_ref[0]
    k = k_ref[0]
    v = v_ref[0]
    s = jax.lax.dot_general(q, k, (((1,), (1,)), ((), ())),
                            preferred_element_type=jnp.float32)
    m = jnp.max(s, axis=-1, keepdims=True)
    S_ = s.shape[1]
    cm = s[:, 0:128]
    for j in range(1, S_ // 128):
        cm = jnp.maximum(cm, s[:, j * 128:(j + 1) * 128])
    lo0 = jnp.min(cm, axis=-1, keepdims=True)
    kcnt = jnp.float32(U)

    def body(_, carry):
        lo, hi = carry
        mid = 0.5 * (lo + hi)
        cnt = jnp.sum((s >= mid).astype(jnp.float32), axis=-1, keepdims=True)
        pred = cnt >= kcnt
        return jnp.where(pred, mid, lo), jnp.where(pred, hi, mid)

    lo, _ = jax.lax.fori_loop(0, n_iters, body, (lo0, m))
    mprime = jnp.maximum(m, 0.0)
    bg = jnp.exp(-mprime)
    p = jnp.where(s >= lo, jnp.exp(s - mprime), bg)
    z = jnp.sum(p, axis=-1, keepdims=True)
    o = jax.lax.dot_general(p, v, (((1,), (0,)), ((), ())),
                            preferred_element_type=jnp.float32)
    o_ref[...] = (o / z)[None]


def _attention(q, k, v, U, l_blk):
    H, L, d = q.shape
    S = k.shape[1]
    return pl.pallas_call(
        functools.partial(_attn_kernel, U=U, n_iters=_N_ITERS),
        grid=(H, L // l_blk),
        in_specs=[
            pl.BlockSpec((1, l_blk, d), lambda h, l: (h, l, 0)),
            pl.BlockSpec((1, S, d), lambda h, l: (h, 0, 0)),
            pl.BlockSpec((1, S, d), lambda h, l: (h, 0, 0)),
        ],
        out_specs=pl.BlockSpec((1, l_blk, d), lambda h, l: (h, l, 0)),
        out_shape=jax.ShapeDtypeStruct((H, L, d), jnp.float32),
    )(q, k, v)


def kernel(queries, keys, values, Wq, bq, Wk, bk, Wv, bv):
    B_, L, D = queries.shape
    S = keys.shape[1]
    U = int(_FACTOR * math.log(L))
    n_blk = min(256, L)
    q = _project(queries.reshape(B_ * L, D), Wq, bq, n_blk)
    k = _project(keys.reshape(B_ * S, D), Wk, bk, n_blk)
    v = _project(values.reshape(B_ * S, D), Wv, bv, n_blk)
    out = _attention(q, k, v, U, n_blk)
    return out.transpose(1, 0, 2).reshape(B_, L, D)


# speedup vs baseline: 2.3430x; 1.4515x over previous
import functools
import math

import jax
import jax.numpy as jnp
from jax.experimental import pallas as pl

N_HEADS = 16
_FACTOR = 5
_N_ITERS = 11


def _proj_kernel(x_ref, w_ref, b_ref, o_ref):
    acc = jax.lax.dot_general(x_ref[...], w_ref[...], (((1,), (1,)), ((), ())),
                              preferred_element_type=jnp.float32)
    o_ref[...] = acc + b_ref[0]


def _project(x, W, b, c_blk):
    N, D = x.shape
    b3 = b.reshape(D // c_blk, 1, c_blk)
    return pl.pallas_call(
        _proj_kernel,
        grid=(D // c_blk,),
        in_specs=[
            pl.BlockSpec((N, D), lambda c: (0, 0)),
            pl.BlockSpec((c_blk, D), lambda c: (c, 0)),
            pl.BlockSpec((1, 1, c_blk), lambda c: (c, 0, 0)),
        ],
        out_specs=pl.BlockSpec((N, c_blk), lambda c: (0, c)),
        out_shape=jax.ShapeDtypeStruct((N, D), jnp.float32),
    )(x, W, b3)


def _attn_kernel(q_ref, k_ref, v_ref, o_ref, *, U, n_iters):
    q = q_ref[...]
    k = k_ref[...]
    v = v_ref[...]
    s = jax.lax.dot_general(q, k, (((1,), (1,)), ((), ())),
                            preferred_element_type=jnp.float32)
    m = jnp.max(s, axis=-1, keepdims=True)
    S_ = s.shape[1]
    cm = s[:, 0:128]
    for j in range(1, S_ // 128):
        cm = jnp.maximum(cm, s[:, j * 128:(j + 1) * 128])
    lo0 = jnp.min(cm, axis=-1, keepdims=True)
    kcnt = jnp.float32(U)

    def body(_, carry):
        lo, hi = carry
        mid = 0.5 * (lo + hi)
        cnt = jnp.sum((s >= mid).astype(jnp.float32), axis=-1, keepdims=True)
        pred = cnt >= kcnt
        return jnp.where(pred, mid, lo), jnp.where(pred, hi, mid)

    lo, _ = jax.lax.fori_loop(0, n_iters, body, (lo0, m))
    mprime = jnp.maximum(m, 0.0)
    bg = jnp.exp(-mprime)
    p = jnp.where(s >= lo, jnp.exp(s - mprime), bg)
    z = jnp.sum(p, axis=-1, keepdims=True)
    o = jax.lax.dot_general(p, v, (((1,), (0,)), ((), ())),
                            preferred_element_type=jnp.float32)
    o_ref[...] = o / z


def _attention(q, k, v, U, l_blk, H):
    L, D = q.shape
    S = k.shape[0]
    d = D // H
    return pl.pallas_call(
        functools.partial(_attn_kernel, U=U, n_iters=_N_ITERS),
        grid=(H, L // l_blk),
        in_specs=[
            pl.BlockSpec((l_blk, d), lambda h, l: (l, h)),
            pl.BlockSpec((S, d), lambda h, l: (0, h)),
            pl.BlockSpec((S, d), lambda h, l: (0, h)),
        ],
        out_specs=pl.BlockSpec((l_blk, d), lambda h, l: (l, h)),
        out_shape=jax.ShapeDtypeStruct((L, D), jnp.float32),
    )(q, k, v)


def kernel(queries, keys, values, Wq, bq, Wk, bk, Wv, bv):
    B_, L, D = queries.shape
    S = keys.shape[1]
    U = int(_FACTOR * math.log(L))
    q = _project(queries.reshape(B_ * L, D), Wq, bq, 128)
    k = _project(keys.reshape(B_ * S, D), Wk, bk, 128)
    v = _project(values.reshape(B_ * S, D), Wv, bv, 128)
    out = _attention(q, k, v, U, min(256, L), N_HEADS)
    return out.reshape(B_, L, D)


# speedup vs baseline: 2.5811x; 1.1016x over previous
import functools
import math

import jax
import jax.numpy as jnp
from jax.experimental import pallas as pl

N_HEADS = 16
_FACTOR = 5
_N_ITERS = 9


def _proj_kernel(x_ref, w_ref, b_ref, o_ref):
    acc = jax.lax.dot_general(x_ref[...], w_ref[...], (((1,), (1,)), ((), ())),
                              preferred_element_type=jnp.float32)
    o_ref[...] = acc + b_ref[0]


def _project(x, W, b, c_blk):
    N, D = x.shape
    b3 = b.reshape(D // c_blk, 1, c_blk)
    return pl.pallas_call(
        _proj_kernel,
        grid=(D // c_blk,),
        in_specs=[
            pl.BlockSpec((N, D), lambda c: (0, 0)),
            pl.BlockSpec((c_blk, D), lambda c: (c, 0)),
            pl.BlockSpec((1, 1, c_blk), lambda c: (c, 0, 0)),
        ],
        out_specs=pl.BlockSpec((N, c_blk), lambda c: (0, c)),
        out_shape=jax.ShapeDtypeStruct((N, D), jnp.float32),
    )(x, W, b3)


def _attn_kernel(q_ref, k_ref, v_ref, o_ref, *, U, n_iters):
    q = q_ref[...]
    k = k_ref[...]
    v = v_ref[...]
    s = jax.lax.dot_general(q, k, (((1,), (1,)), ((), ())),
                            preferred_element_type=jnp.float32)
    m = jnp.max(s, axis=-1, keepdims=True)
    S_ = s.shape[1]
    cm = s[:, 0:128]
    for j in range(1, S_ // 128):
        cm = jnp.maximum(cm, s[:, j * 128:(j + 1) * 128])
    lo0 = jnp.min(cm, axis=-1, keepdims=True)
    kcnt = jnp.float32(U)

    def body(_, carry):
        lo, hi = carry
        mid = 0.5 * (lo + hi)
        cnt = jnp.sum((s >= mid).astype(jnp.float32), axis=-1, keepdims=True)
        pred = cnt >= kcnt
        return jnp.where(pred, mid, lo), jnp.where(pred, hi, mid)

    lo, _ = jax.lax.fori_loop(0, n_iters, body, (lo0, m))
    mprime = jnp.maximum(m, 0.0)
    bg = jnp.exp(-mprime)
    p = jnp.where(s >= lo, jnp.exp(s - mprime), bg).astype(jnp.bfloat16)
    v1 = jnp.concatenate(
        [v.astype(jnp.bfloat16),
         jnp.ones((v.shape[0], 128), jnp.bfloat16)], axis=1)
    oz = jax.lax.dot_general(p, v1, (((1,), (0,)), ((), ())),
                             preferred_element_type=jnp.float32)
    o_ref[...] = oz[:, :v.shape[1]] / oz[:, v.shape[1]:v.shape[1] + 1]


def _attention(q, k, v, U, l_blk, H):
    L, D = q.shape
    S = k.shape[0]
    d = D // H
    return pl.pallas_call(
        functools.partial(_attn_kernel, U=U, n_iters=_N_ITERS),
        grid=(H, L // l_blk),
        in_specs=[
            pl.BlockSpec((l_blk, d), lambda h, l: (l, h)),
            pl.BlockSpec((S, d), lambda h, l: (0, h)),
            pl.BlockSpec((S, d), lambda h, l: (0, h)),
        ],
        out_specs=pl.BlockSpec((l_blk, d), lambda h, l: (l, h)),
        out_shape=jax.ShapeDtypeStruct((L, D), jnp.float32),
    )(q, k, v)


def kernel(queries, keys, values, Wq, bq, Wk, bk, Wv, bv):
    B_, L, D = queries.shape
    S = keys.shape[1]
    U = int(_FACTOR * math.log(L))
    q = _project(queries.reshape(B_ * L, D), Wq, bq, 128)
    k = _project(keys.reshape(B_ * S, D), Wk, bk, 128)
    v = _project(values.reshape(B_ * S, D), Wv, bv, 128)
    out = _attention(q, k, v, U, min(256, L), N_HEADS)
    return out.reshape(B_, L, D)


# speedup vs baseline: 2.8898x; 1.1196x over previous
import functools
import math

import jax
import jax.numpy as jnp
from jax.experimental import pallas as pl

N_HEADS = 16
_FACTOR = 5
_N_ITERS = 9


def _proj_kernel(x_ref, w_ref, b_ref, o_ref):
    acc = jax.lax.dot_general(x_ref[...], w_ref[...], (((1,), (1,)), ((), ())),
                              preferred_element_type=jnp.float32)
    o_ref[...] = acc + b_ref[0]


def _project(x, W, b, c_blk):
    N, D = x.shape
    b3 = b.reshape(D // c_blk, 1, c_blk)
    return pl.pallas_call(
        _proj_kernel,
        grid=(D // c_blk,),
        in_specs=[
            pl.BlockSpec((N, D), lambda c: (0, 0)),
            pl.BlockSpec((c_blk, D), lambda c: (c, 0)),
            pl.BlockSpec((1, 1, c_blk), lambda c: (c, 0, 0)),
        ],
        out_specs=pl.BlockSpec((N, c_blk), lambda c: (0, c)),
        out_shape=jax.ShapeDtypeStruct((N, D), jnp.float32),
    )(x, W, b3)


def _attn_kernel(q_ref, k_ref, v_ref, o_ref, *, U, n_iters):
    q = q_ref[...]
    k = k_ref[...]
    v = v_ref[...]
    s = jax.lax.dot_general(q, k, (((1,), (1,)), ((), ())),
                            preferred_element_type=jnp.float32)
    S_ = s.shape[1]
    cm = s[:, 0:128]
    for j in range(1, S_ // 128):
        cm = jnp.maximum(cm, s[:, j * 128:(j + 1) * 128])
    m = jnp.max(cm, axis=-1, keepdims=True)
    lo0 = jnp.min(cm, axis=-1, keepdims=True)
    kcnt = jnp.float32(U)

    def body(_, carry):
        lo, hi = carry
        mid = 0.5 * (lo + hi)
        cnt = jnp.sum((s >= mid).astype(jnp.float32), axis=-1, keepdims=True)
        pred = cnt >= kcnt
        return jnp.where(pred, mid, lo), jnp.where(pred, hi, mid)

    lo, _ = jax.lax.fori_loop(0, n_iters, body, (lo0, m))
    mprime = jnp.maximum(m, 0.0)
    bg = jnp.exp(-mprime)
    p = jnp.where(s >= lo, jnp.exp(s - mprime), bg).astype(jnp.bfloat16)
    v1 = jnp.concatenate(
        [v.astype(jnp.bfloat16),
         jnp.ones((v.shape[0], 128), jnp.bfloat16)], axis=1)
    oz = jax.lax.dot_general(p, v1, (((1,), (0,)), ((), ())),
                             preferred_element_type=jnp.float32)
    o_ref[...] = oz[:, :v.shape[1]] / oz[:, v.shape[1]:v.shape[1] + 1]


def _attention(q, k, v, U, l_blk, H):
    L, D = q.shape
    S = k.shape[0]
    d = D // H
    return pl.pallas_call(
        functools.partial(_attn_kernel, U=U, n_iters=_N_ITERS),
        grid=(H, L // l_blk),
        in_specs=[
            pl.BlockSpec((l_blk, d), lambda h, l: (l, h)),
            pl.BlockSpec((S, d), lambda h, l: (0, h)),
            pl.BlockSpec((S, d), lambda h, l: (0, h)),
        ],
        out_specs=pl.BlockSpec((l_blk, d), lambda h, l: (l, h)),
        out_shape=jax.ShapeDtypeStruct((L, D), jnp.float32),
    )(q, k, v)


def kernel(queries, keys, values, Wq, bq, Wk, bk, Wv, bv):
    B_, L, D = queries.shape
    S = keys.shape[1]
    U = int(_FACTOR * math.log(L))
    q = _project(queries.reshape(B_ * L, D), Wq, bq, 128)
    k = _project(keys.reshape(B_ * S, D), Wk, bk, 128)
    v = _project(values.reshape(B_ * S, D), Wv, bv, 128)
    out = _attention(q, k, v, U, min(512, L), N_HEADS)
    return out.reshape(B_, L, D)


# speedup vs baseline: 3.0082x; 1.0410x over previous
import functools
import math

import jax
import jax.numpy as jnp
from jax.experimental import pallas as pl

N_HEADS = 16
_FACTOR = 5
_N_ITERS = 9


def _proj_kernel(x_ref, w_ref, b_ref, o_ref):
    acc = jax.lax.dot_general(x_ref[...], w_ref[...], (((1,), (1,)), ((), ())),
                              preferred_element_type=jnp.float32)
    o_ref[...] = acc + b_ref[0]


def _project(x, W, b, c_blk):
    N, D = x.shape
    b3 = b.reshape(D // c_blk, 1, c_blk)
    return pl.pallas_call(
        _proj_kernel,
        grid=(D // c_blk,),
        in_specs=[
            pl.BlockSpec((N, D), lambda c: (0, 0)),
            pl.BlockSpec((c_blk, D), lambda c: (c, 0)),
            pl.BlockSpec((1, 1, c_blk), lambda c: (c, 0, 0)),
        ],
        out_specs=pl.BlockSpec((N, c_blk), lambda c: (0, c)),
        out_shape=jax.ShapeDtypeStruct((N, D), jnp.float32),
    )(x, W, b3)


def _attn_kernel(q_ref, k_ref, v_ref, o_ref, *, U, n_iters):
    q = q_ref[...]
    k = k_ref[...]
    v = v_ref[...]
    s = jax.lax.dot_general(q, k, (((1,), (1,)), ((), ())),
                            preferred_element_type=jnp.float32)
    S_ = s.shape[1]
    cm = s[:, 0:128]
    for j in range(1, S_ // 128):
        cm = jnp.maximum(cm, s[:, j * 128:(j + 1) * 128])
    m = jnp.max(cm, axis=-1, keepdims=True)
    lo0 = jnp.min(cm, axis=-1, keepdims=True)
    kcnt = jnp.float32(U)

    def body(_, carry):
        lo, hi = carry
        mid = 0.5 * (lo + hi)
        cnt = jnp.sum((s >= mid).astype(jnp.float32), axis=-1, keepdims=True)
        pred = cnt >= kcnt
        return jnp.where(pred, mid, lo), jnp.where(pred, hi, mid)

    lo, _ = jax.lax.fori_loop(0, n_iters, body, (lo0, m))
    mprime = jnp.maximum(m, 0.0)
    bg = jnp.exp(-mprime)
    p = jnp.where(s >= lo, jnp.exp(s - mprime), bg).astype(jnp.bfloat16)
    v1 = jnp.concatenate(
        [v.astype(jnp.bfloat16),
         jnp.ones((v.shape[0], 128), jnp.bfloat16)], axis=1)
    oz = jax.lax.dot_general(p, v1, (((1,), (0,)), ((), ())),
                             preferred_element_type=jnp.float32)
    o_ref[...] = oz[:, :v.shape[1]] / oz[:, v.shape[1]:v.shape[1] + 1]


def _attention(q, k, v, U, l_blk, H):
    L, D = q.shape
    S = k.shape[0]
    d = D // H
    return pl.pallas_call(
        functools.partial(_attn_kernel, U=U, n_iters=_N_ITERS),
        grid=(H, L // l_blk),
        in_specs=[
            pl.BlockSpec((l_blk, d), lambda h, l: (l, h)),
            pl.BlockSpec((S, d), lambda h, l: (0, h)),
            pl.BlockSpec((S, d), lambda h, l: (0, h)),
        ],
        out_specs=pl.BlockSpec((l_blk, d), lambda h, l: (l, h)),
        out_shape=jax.ShapeDtypeStruct((L, D), jnp.float32),
    )(q, k, v)


def kernel(queries, keys, values, Wq, bq, Wk, bk, Wv, bv):
    B_, L, D = queries.shape
    S = keys.shape[1]
    U = int(_FACTOR * math.log(L))
    q = _project(queries.reshape(B_ * L, D), Wq, bq, 128)
    k = _project(keys.reshape(B_ * S, D), Wk, bk, 128)
    v = _project(values.reshape(B_ * S, D), Wv, bv, 128)
    out = _attention(q, k, v, U, min(1024, L), N_HEADS)
    return out.reshape(B_, L, D)
